# Initial kernel scaffold; baseline (speedup 1.0000x reference)
import jax
import jax.numpy as jnp
from jax.experimental import pallas as pl


def kernel(x):
    raise NotImplementedError("write your pallas kernel here")


# speedup vs baseline: 35.4202x; 35.4202x over previous
import functools

import jax
import jax.numpy as jnp
from jax import lax
from jax.experimental import pallas as pl
from jax.experimental.pallas import tpu as pltpu
from jax.experimental.pallas import tpu_sc as plsc

Q_LOW_ = 0.005
Q_HIGH_ = 0.995
EPS_ = 1e-06

B_, C_, D_, H_, W_ = 2, 4, 96, 128, 128
NCH = B_ * C_
N = D_ * H_ * W_

L = 16
NC = 2
NS = 16
NW = NC * NS
WPC = NW // NCH
PER_W = N // WPC
CHUNK = 16384
NCHUNKS = PER_W // CHUNK
UNROLL = 4

SHIFT1 = 20
BINS1 = 4096
SHIFT2 = 10
BINS2 = 1024
NT = 4

_K_LO = int(Q_LOW_ * (N - 1))
_K_HI = int(Q_HIGH_ * (N - 1))
_FRAC_LO = Q_LOW_ * (N - 1) - _K_LO
_FRAC_HI = Q_HIGH_ * (N - 1) - _K_HI
_KS = (_K_LO, _K_LO + 1, _K_HI, _K_HI + 1)

_mesh = plsc.VectorSubcoreMesh(core_axis_name="c", subcore_axis_name="s")


def _worker_id():
    return lax.axis_index("s") * NC + lax.axis_index("c")


def _zero_hist(hist):
    zeros = jnp.zeros((L,), jnp.int32)

    def zbody(i, carry):
        hist[pl.ds(i * L, L)] = zeros
        return carry

    lax.fori_loop(0, (BINS1 * L) // L, zbody, 0)


def _key_of(u):
    return jnp.where(u >= jnp.uint32(1 << 31), ~u, u | jnp.uint32(1 << 31))


@functools.partial(
    pl.kernel,
    out_type=jax.ShapeDtypeStruct((NW, BINS1 * L), jnp.int32),
    mesh=_mesh,
    compiler_params=pltpu.CompilerParams(needs_layout_passes=False),
    scratch_types=[
        pltpu.VMEM((CHUNK,), jnp.uint32),
        pltpu.VMEM((CHUNK,), jnp.uint32),
        pltpu.VMEM((BINS1 * L,), jnp.int32),
        pltpu.SemaphoreType.DMA,
        pltpu.SemaphoreType.DMA,
    ],
)
def _hist1(x_hbm, out_hbm, buf0, buf1, hist, sem0, sem1):
    wid = _worker_id()
    ch = wid // WPC
    base = (wid % WPC) * PER_W
    _zero_hist(hist)

    lane = lax.iota(jnp.int32, L).astype(jnp.uint32)
    ones = jnp.ones((L,), jnp.int32)
    sems = (sem0, sem1)
    bufs = (buf0, buf1)

    cps = [None, None]
    cps[0] = pltpu.async_copy(x_hbm.at[ch, pl.ds(base, CHUNK)], bufs[0], sems[0])
    for j in range(NCHUNKS):
        cur = j % 2
        nxt = (j + 1) % 2
        if j + 1 < NCHUNKS:
            cps[nxt] = pltpu.async_copy(
                x_hbm.at[ch, pl.ds(base + (j + 1) * CHUNK, CHUNK)],
                bufs[nxt], sems[nxt])
        cps[cur].wait()
        bufc = bufs[cur]

        def body(i, carry):
            for r in range(UNROLL):
                off = (i * UNROLL + r) * L
                key = _key_of(bufc[pl.ds(off, L)])
                b = key >> jnp.uint32(SHIFT1)
                idx = (b << jnp.uint32(4)) | lane
                plsc.addupdate_scatter(hist, [idx.astype(jnp.int32)], ones)
            return carry

        lax.fori_loop(0, CHUNK // (L * UNROLL), body, 0)

    pltpu.sync_copy(hist, out_hbm.at[wid])


@functools.partial(
    pl.kernel,
    out_type=jax.ShapeDtypeStruct((NW, NT * BINS2 * L), jnp.int32),
    mesh=_mesh,
    compiler_params=pltpu.CompilerParams(needs_layout_passes=False),
    scratch_types=[
        pltpu.VMEM((CHUNK,), jnp.uint32),
        pltpu.VMEM((CHUNK,), jnp.uint32),
        pltpu.VMEM((NT * BINS2 * L,), jnp.int32),
        pltpu.VMEM((NT, L), jnp.uint32),
        pltpu.SemaphoreType.DMA,
        pltpu.SemaphoreType.DMA,
    ],
)
def _hist2(x_hbm, tgt_hbm, out_hbm, buf0, buf1, hist, tbuf, sem0, sem1):
    wid = _worker_id()
    ch = wid // WPC
    base = (wid % WPC) * PER_W
    _zero_hist(hist)
    pltpu.sync_copy(tgt_hbm.at[ch], tbuf)
    t0 = tbuf[0, :]
    t1 = tbuf[1, :]
    t2 = tbuf[2, :]
    t3 = tbuf[3, :]

    lane = lax.iota(jnp.int32, L).astype(jnp.uint32)
    ones = jnp.ones((L,), jnp.int32)
    sems = (sem0, sem1)
    bufs = (buf0, buf1)

    cps = [None, None]
    cps[0] = pltpu.async_copy(x_hbm.at[ch, pl.ds(base, CHUNK)], bufs[0], sems[0])
    for j in range(NCHUNKS):
        cur = j % 2
        nxt = (j + 1) % 2
        if j + 1 < NCHUNKS:
            cps[nxt] = pltpu.async_copy(
                x_hbm.at[ch, pl.ds(base + (j + 1) * CHUNK, CHUNK)],
                bufs[nxt], sems[nxt])
        cps[cur].wait()
        bufc = bufs[cur]

        def body(i, carry):
            for r in range(UNROLL):
                off = (i * UNROLL + r) * L
                key = _key_of(bufc[pl.ds(off, L)])
                pref = key >> jnp.uint32(SHIFT1)
                sub = (key >> jnp.uint32(SHIFT2)) & jnp.uint32(BINS2 - 1)
                s0 = pref == t0
                s1 = pref == t1
                s2 = pref == t2
                s3 = pref == t3
                m_any = (s0 | s1) | (s2 | s3)
                slot = jnp.where(
                    s0, jnp.uint32(0),
                    jnp.where(s1, jnp.uint32(1),
                              jnp.where(s2, jnp.uint32(2), jnp.uint32(3))))
                idx = ((((slot << jnp.uint32(10)) | sub) << jnp.uint32(4)) | lane)
                plsc.addupdate_scatter(
                    hist, [idx.astype(jnp.int32)], ones, mask=m_any)
            return carry

        lax.fori_loop(0, CHUNK // (L * UNROLL), body, 0)

    pltpu.sync_copy(hist, out_hbm.at[wid])


def _unmap_key(key_u32):
    top = key_u32 & jnp.uint32(1 << 31)
    u = jnp.where(top != 0, key_u32 ^ jnp.uint32(1 << 31), ~key_u32)
    return lax.bitcast_convert_type(u, jnp.float32)


def _sums_body(q_ref, x_ref, s_ref, ss_ref):
    c = pl.program_id(0)
    xc = jnp.clip(x_ref[0], q_ref[0, c], q_ref[1, c])
    ps = jnp.sum(xc, axis=0)
    pss = jnp.sum(xc * xc, axis=0)
    s_ref[0, 0] = jnp.broadcast_to(ps[None, :], (8, 128))
    ss_ref[0, 0] = jnp.broadcast_to(pss[None, :], (8, 128))


def _norm_body(sc_ref, x_ref, o_ref):
    c = pl.program_id(0)
    xc = jnp.clip(x_ref[0], sc_ref[0, c], sc_ref[1, c])
    o_ref[0] = (xc - sc_ref[2, c]) * sc_ref[3, c]


def kernel(x):
    x2 = lax.bitcast_convert_type(x.reshape(NCH, N), jnp.uint32)

    h1_raw = _hist1(x2)
    h1 = h1_raw.reshape(NCH, WPC, BINS1, L).sum(axis=(1, 3))
    f1 = jnp.cumsum(h1, axis=1)

    ks = jnp.array(_KS, jnp.int32)
    bins = jax.vmap(lambda f: jnp.searchsorted(f, ks, side="right"))(f1)
    f1p = jnp.pad(f1, ((0, 0), (1, 0)))
    below = jnp.take_along_axis(f1p, bins, axis=1)
    rank = ks[None, :] - below

    tgt = jnp.broadcast_to(
        bins.astype(jnp.uint32)[:, :, None], (NCH, NT, L))
    h2_raw = _hist2(x2, tgt)
    h2_all = h2_raw.reshape(NCH, WPC, NT, BINS2, L).sum(axis=(1, 4))

    eq = bins[:, None, :] == bins[:, :, None]
    fm = jnp.argmax(eq, axis=2)
    h2 = jnp.take_along_axis(h2_all, fm[:, :, None], axis=1)
    f2 = jnp.cumsum(h2, axis=2)
    sub = jax.vmap(jax.vmap(
        lambda f, r: jnp.searchsorted(f, r, side="right")))(f2, rank)
    f2p = jnp.pad(f2, ((0, 0), (0, 0), (1, 0)))
    below2 = jnp.take_along_axis(f2p, sub[:, :, None], axis=2)[..., 0]
    rank2 = (rank - below2).astype(jnp.float32)
    cnt = jnp.take_along_axis(h2, sub[:, :, None], axis=2)[..., 0]
    cnt = jnp.maximum(cnt, 1).astype(jnp.float32)

    key_lo = (bins.astype(jnp.uint32) << jnp.uint32(SHIFT1)) | (
        sub.astype(jnp.uint32) << jnp.uint32(SHIFT2))
    key_hi = key_lo + jnp.uint32((1 << SHIFT2) - 1)
    v_lo = _unmap_key(key_lo)
    v_hi = _unmap_key(key_hi)
    v = v_lo + ((rank2 + 0.5) / cnt) * (v_hi - v_lo)

    q_lo = v[:, 0] * (1.0 - _FRAC_LO) + v[:, 1] * _FRAC_LO
    q_hi = v[:, 2] * (1.0 - _FRAC_HI) + v[:, 3] * _FRAC_HI

    x3 = x.reshape(NCH, N // 128, 128)
    g3 = 12
    bs3 = (N // 128) // g3
    sums, sumsqs = pl.pallas_call(
        _sums_body,
        grid=(NCH, g3),
        in_specs=[
            pl.BlockSpec(memory_space=pltpu.SMEM),
            pl.BlockSpec((1, bs3, 128), lambda c, i: (c, i, 0)),
        ],
        out_specs=[
            pl.BlockSpec((1, 1, 8, 128), lambda c, i: (c, i, 0, 0)),
            pl.BlockSpec((1, 1, 8, 128), lambda c, i: (c, i, 0, 0)),
        ],
        out_shape=[
            jax.ShapeDtypeStruct((NCH, g3, 8, 128), jnp.float32),
            jax.ShapeDtypeStruct((NCH, g3, 8, 128), jnp.float32),
        ],
    )(jnp.stack([q_lo, q_hi]), x3)

    s = sums[:, :, 0, :].sum(axis=(1, 2))
    ss = sumsqs[:, :, 0, :].sum(axis=(1, 2))
    mean = s / N
    var = (ss - s * s / N) / (N - 1)
    std = jnp.maximum(jnp.sqrt(jnp.maximum(var, 0.0)), EPS_)
    inv = 1.0 / std

    out = pl.pallas_call(
        _norm_body,
        grid=(NCH, g3),
        in_specs=[
            pl.BlockSpec(memory_space=pltpu.SMEM),
            pl.BlockSpec((1, bs3, 128), lambda c, i: (c, i, 0)),
        ],
        out_specs=pl.BlockSpec((1, bs3, 128), lambda c, i: (c, i, 0)),
        out_shape=jax.ShapeDtypeStruct((NCH, N // 128, 128), jnp.float32),
    )(jnp.stack([q_lo, q_hi, mean, inv]), x3)

    return out.reshape(x.shape)


# speedup vs baseline: 68.3195x; 1.9288x over previous
import functools

import jax
import jax.numpy as jnp
from jax import lax
from jax.experimental import pallas as pl
from jax.experimental.pallas import tpu as pltpu
from jax.experimental.pallas import tpu_sc as plsc

Q_LOW_ = 0.005
Q_HIGH_ = 0.995
EPS_ = 1e-06

B_, C_, D_, H_, W_ = 2, 4, 96, 128, 128
NCH = B_ * C_
N = D_ * H_ * W_

L = 16
NC = 2
NS = 16
NW = NC * NS
WPC = NW // NCH
PER_W = N // WPC
CHUNK = 24576
NCHUNKS = PER_W // CHUNK
UNROLL = 8

SHIFT1 = 16
BINS1 = 1 << (32 - SHIFT1)

_K_LO = int(Q_LOW_ * (N - 1))
_K_HI = int(Q_HIGH_ * (N - 1))
_FRAC_LO = Q_LOW_ * (N - 1) - _K_LO
_FRAC_HI = Q_HIGH_ * (N - 1) - _K_HI
_KS = (_K_LO, _K_LO + 1, _K_HI, _K_HI + 1)

_mesh = plsc.VectorSubcoreMesh(core_axis_name="c", subcore_axis_name="s")


def _key_of(u):
    return jnp.where(u >= jnp.uint32(1 << 31), ~u, u | jnp.uint32(1 << 31))


@functools.partial(
    pl.kernel,
    out_type=jax.ShapeDtypeStruct((NW, BINS1), jnp.int32),
    mesh=_mesh,
    compiler_params=pltpu.CompilerParams(needs_layout_passes=False),
    scratch_types=[
        pltpu.VMEM((CHUNK,), jnp.uint32),
        pltpu.VMEM((CHUNK,), jnp.uint32),
        pltpu.VMEM((BINS1,), jnp.int32),
        pltpu.SemaphoreType.DMA,
        pltpu.SemaphoreType.DMA,
    ],
)
def _hist1(x_hbm, out_hbm, buf0, buf1, hist, sem0, sem1):
    wid = lax.axis_index("s") * NC + lax.axis_index("c")
    ch = wid // WPC
    base = (wid % WPC) * PER_W

    zeros = jnp.zeros((L,), jnp.int32)

    def zbody(i, carry):
        hist[pl.ds(i * L, L)] = zeros
        return carry

    lax.fori_loop(0, BINS1 // L, zbody, 0)

    ones = jnp.ones((L,), jnp.int32)
    sems = (sem0, sem1)
    bufs = (buf0, buf1)

    cps = [None, None]
    cps[0] = pltpu.async_copy(x_hbm.at[ch, pl.ds(base, CHUNK)], bufs[0], sems[0])
    for j in range(NCHUNKS):
        cur = j % 2
        nxt = (j + 1) % 2
        if j + 1 < NCHUNKS:
            cps[nxt] = pltpu.async_copy(
                x_hbm.at[ch, pl.ds(base + (j + 1) * CHUNK, CHUNK)],
                bufs[nxt], sems[nxt])
        cps[cur].wait()
        bufc = bufs[cur]

        def body(i, carry):
            for r in range(UNROLL):
                off = (i * UNROLL + r) * L
                key = _key_of(bufc[pl.ds(off, L)])
                b = key >> jnp.uint32(SHIFT1)
                plsc.addupdate_scatter(hist, [b.astype(jnp.int32)], ones)
            return carry

        lax.fori_loop(0, CHUNK // (L * UNROLL), body, 0)

    pltpu.sync_copy(hist, out_hbm.at[wid])


def _unmap_key(key_u32):
    top = key_u32 & jnp.uint32(1 << 31)
    u = jnp.where(top != 0, key_u32 ^ jnp.uint32(1 << 31), ~key_u32)
    return lax.bitcast_convert_type(u, jnp.float32)


def _sums_body(q_ref, x_ref, s_ref, ss_ref):
    c = pl.program_id(0)
    xc = jnp.clip(x_ref[0], q_ref[0, c], q_ref[1, c])
    ps = jnp.sum(xc, axis=0)
    pss = jnp.sum(xc * xc, axis=0)
    s_ref[0, 0] = jnp.broadcast_to(ps[None, :], (8, 128))
    ss_ref[0, 0] = jnp.broadcast_to(pss[None, :], (8, 128))


def _norm_body(sc_ref, x_ref, o_ref):
    c = pl.program_id(0)
    xc = jnp.clip(x_ref[0], sc_ref[0, c], sc_ref[1, c])
    o_ref[0] = (xc - sc_ref[2, c]) * sc_ref[3, c]


def kernel(x):
    x2 = lax.bitcast_convert_type(x.reshape(NCH, N), jnp.uint32)

    h1_raw = _hist1(x2)
    h1 = h1_raw.reshape(NCH, WPC, BINS1).sum(axis=1)
    f1 = jnp.cumsum(h1, axis=1)

    ks = jnp.array(_KS, jnp.int32)
    bins = jax.vmap(lambda f: jnp.searchsorted(f, ks, side="right"))(f1)
    f1p = jnp.pad(f1, ((0, 0), (1, 0)))
    below = jnp.take_along_axis(f1p, bins, axis=1)
    rank = (ks[None, :] - below).astype(jnp.float32)
    cnt = jnp.take_along_axis(h1, bins, axis=1)
    cnt = jnp.maximum(cnt, 1).astype(jnp.float32)

    key_lo = bins.astype(jnp.uint32) << jnp.uint32(SHIFT1)
    key_hi = key_lo + jnp.uint32((1 << SHIFT1) - 1)
    v_lo = _unmap_key(key_lo)
    v_hi = _unmap_key(key_hi)
    v = v_lo + ((rank + 0.5) / cnt) * (v_hi - v_lo)

    q_lo = v[:, 0] * (1.0 - _FRAC_LO) + v[:, 1] * _FRAC_LO
    q_hi = v[:, 2] * (1.0 - _FRAC_HI) + v[:, 3] * _FRAC_HI

    x3 = x.reshape(NCH, N // 128, 128)
    g3 = 12
    bs3 = (N // 128) // g3
    sums, sumsqs = pl.pallas_call(
        _sums_body,
        grid=(NCH, g3),
        in_specs=[
            pl.BlockSpec(memory_space=pltpu.SMEM),
            pl.BlockSpec((1, bs3, 128), lambda c, i: (c, i, 0)),
        ],
        out_specs=[
            pl.BlockSpec((1, 1, 8, 128), lambda c, i: (c, i, 0, 0)),
            pl.BlockSpec((1, 1, 8, 128), lambda c, i: (c, i, 0, 0)),
        ],
        out_shape=[
            jax.ShapeDtypeStruct((NCH, g3, 8, 128), jnp.float32),
            jax.ShapeDtypeStruct((NCH, g3, 8, 128), jnp.float32),
        ],
    )(jnp.stack([q_lo, q_hi]), x3)

    s = sums[:, :, 0, :].sum(axis=(1, 2))
    ss = sumsqs[:, :, 0, :].sum(axis=(1, 2))
    mean = s / N
    var = (ss - s * s / N) / (N - 1)
    std = jnp.maximum(jnp.sqrt(jnp.maximum(var, 0.0)), EPS_)
    inv = 1.0 / std

    out = pl.pallas_call(
        _norm_body,
        grid=(NCH, g3),
        in_specs=[
            pl.BlockSpec(memory_space=pltpu.SMEM),
            pl.BlockSpec((1, bs3, 128), lambda c, i: (c, i, 0)),
        ],
        out_specs=pl.BlockSpec((1, bs3, 128), lambda c, i: (c, i, 0)),
        out_shape=jax.ShapeDtypeStruct((NCH, N // 128, 128), jnp.float32),
    )(jnp.stack([q_lo, q_hi, mean, inv]), x3)

    return out.reshape(x.shape)


# speedup vs baseline: 78.3576x; 1.1469x over previous
import functools

import jax
import jax.numpy as jnp
from jax import lax
from jax.experimental import pallas as pl
from jax.experimental.pallas import tpu as pltpu
from jax.experimental.pallas import tpu_sc as plsc

Q_LOW_ = 0.005
Q_HIGH_ = 0.995
EPS_ = 1e-06

B_, C_, D_, H_, W_ = 2, 4, 96, 128, 128
NCH = B_ * C_
N = D_ * H_ * W_

L = 16
NC = 2
NS = 16
NW = NC * NS
WPC = NW // NCH
PER_W = N // WPC
CHUNK = 24576
NCHUNKS = PER_W // CHUNK
UNROLL = 8

SHIFT1 = 16
BINS1 = 1 << (32 - SHIFT1)

_K_LO = int(Q_LOW_ * (N - 1))
_K_HI = int(Q_HIGH_ * (N - 1))
_FRAC_LO = Q_LOW_ * (N - 1) - _K_LO
_FRAC_HI = Q_HIGH_ * (N - 1) - _K_HI
_KS = (_K_LO, _K_LO + 1, _K_HI, _K_HI + 1)

_mesh = plsc.VectorSubcoreMesh(core_axis_name="c", subcore_axis_name="s")


def _key_of(u):
    return jnp.where(u >= jnp.uint32(1 << 31), ~u, u | jnp.uint32(1 << 31))


@functools.partial(
    pl.kernel,
    out_type=jax.ShapeDtypeStruct((NW, BINS1), jnp.int32),
    mesh=_mesh,
    compiler_params=pltpu.CompilerParams(needs_layout_passes=False),
    scratch_types=[
        pltpu.VMEM((CHUNK,), jnp.uint32),
        pltpu.VMEM((CHUNK,), jnp.uint32),
        pltpu.VMEM((BINS1,), jnp.int32),
        pltpu.SemaphoreType.DMA,
        pltpu.SemaphoreType.DMA,
    ],
)
def _hist1(x_hbm, out_hbm, buf0, buf1, hist, sem0, sem1):
    wid = lax.axis_index("s") * NC + lax.axis_index("c")
    ch = wid // WPC
    base = (wid % WPC) * PER_W

    zeros = jnp.zeros((L,), jnp.int32)

    def zbody(i, carry):
        hist[pl.ds(i * L, L)] = zeros
        return carry

    lax.fori_loop(0, BINS1 // L, zbody, 0)

    ones = jnp.ones((L,), jnp.int32)
    sems = (sem0, sem1)
    bufs = (buf0, buf1)

    cps = [None, None]
    cps[0] = pltpu.async_copy(x_hbm.at[ch, pl.ds(base, CHUNK)], bufs[0], sems[0])
    for j in range(NCHUNKS):
        cur = j % 2
        nxt = (j + 1) % 2
        if j + 1 < NCHUNKS:
            cps[nxt] = pltpu.async_copy(
                x_hbm.at[ch, pl.ds(base + (j + 1) * CHUNK, CHUNK)],
                bufs[nxt], sems[nxt])
        cps[cur].wait()
        bufc = bufs[cur]

        def body(i, carry):
            for r in range(UNROLL):
                off = (i * UNROLL + r) * L
                key = _key_of(bufc[pl.ds(off, L)])
                b = key >> jnp.uint32(SHIFT1)
                plsc.addupdate_scatter(hist, [b.astype(jnp.int32)], ones)
            return carry

        lax.fori_loop(0, CHUNK // (L * UNROLL), body, 0)

    pltpu.sync_copy(hist, out_hbm.at[wid])


def _unmap_key(key_u32):
    top = key_u32 & jnp.uint32(1 << 31)
    u = jnp.where(top != 0, key_u32 ^ jnp.uint32(1 << 31), ~key_u32)
    return lax.bitcast_convert_type(u, jnp.float32)


def _norm_body(sc_ref, x_ref, o_ref):
    c = pl.program_id(0)
    xc = jnp.clip(x_ref[0], sc_ref[0, c], sc_ref[1, c])
    o_ref[0] = (xc - sc_ref[2, c]) * sc_ref[3, c]


def kernel(x):
    x2 = lax.bitcast_convert_type(x.reshape(NCH, N), jnp.uint32)

    h1_raw = _hist1(x2)
    h1 = h1_raw.reshape(NCH, WPC, BINS1).sum(axis=1)
    f1 = jnp.cumsum(h1, axis=1)

    ks = jnp.array(_KS, jnp.int32)
    bins = jax.vmap(lambda f: jnp.searchsorted(f, ks, side="right"))(f1)
    f1p = jnp.pad(f1, ((0, 0), (1, 0)))
    below = jnp.take_along_axis(f1p, bins, axis=1)
    rank = (ks[None, :] - below).astype(jnp.float32)
    cnt = jnp.take_along_axis(h1, bins, axis=1)
    cnt = jnp.maximum(cnt, 1).astype(jnp.float32)

    key_lo = bins.astype(jnp.uint32) << jnp.uint32(SHIFT1)
    key_hi = key_lo + jnp.uint32((1 << SHIFT1) - 1)
    v_lo = _unmap_key(key_lo)
    v_hi = _unmap_key(key_hi)
    v = v_lo + ((rank + 0.5) / cnt) * (v_hi - v_lo)

    q_lo = v[:, 0] * (1.0 - _FRAC_LO) + v[:, 1] * _FRAC_LO
    q_hi = v[:, 2] * (1.0 - _FRAC_HI) + v[:, 3] * _FRAC_HI

    all_bins = jnp.arange(BINS1, dtype=jnp.uint32)
    e_lo = jnp.nan_to_num(_unmap_key(all_bins << jnp.uint32(SHIFT1)),
                          posinf=3e38, neginf=-3e38)
    e_hi = jnp.nan_to_num(
        _unmap_key((all_bins << jnp.uint32(SHIFT1))
                   + jnp.uint32((1 << SHIFT1) - 1)),
        posinf=3e38, neginf=-3e38)
    lo = jnp.clip(e_lo[None, :], q_lo[:, None], q_hi[:, None])
    hi = jnp.clip(e_hi[None, :], q_lo[:, None], q_hi[:, None])
    hf = h1.astype(jnp.float32)
    s = jnp.sum(hf * ((lo + hi) * 0.5), axis=1)
    ss = jnp.sum(hf * ((lo * lo + lo * hi + hi * hi) * (1.0 / 3.0)), axis=1)
    mean = s / N
    var = (ss - s * s / N) / (N - 1)
    std = jnp.maximum(jnp.sqrt(jnp.maximum(var, 0.0)), EPS_)
    inv = 1.0 / std

    x3 = x.reshape(NCH, N // 128, 128)
    g3 = 12
    bs3 = (N // 128) // g3
    out = pl.pallas_call(
        _norm_body,
        grid=(NCH, g3),
        in_specs=[
            pl.BlockSpec(memory_space=pltpu.SMEM),
            pl.BlockSpec((1, bs3, 128), lambda c, i: (c, i, 0)),
        ],
        out_specs=pl.BlockSpec((1, bs3, 128), lambda c, i: (c, i, 0)),
        out_shape=jax.ShapeDtypeStruct((NCH, N // 128, 128), jnp.float32),
    )(jnp.stack([q_lo, q_hi, mean, inv]), x3)

    return out.reshape(x.shape)
